# baseline (speedup 1.0000x reference)
import functools

import jax
import jax.numpy as jnp
from jax import lax
from jax.experimental import pallas as pl
from jax.experimental.pallas import tpu as pltpu
from jax.experimental.pallas import tpu_sc as plsc

_NC, _NS, _L = 2, 16, 16
_NW = _NC * _NS


def _dist_argmin_body(K, TK, batch_ref, keys_ref, out_ref, minval, minidx):
    c = pl.program_id(0)
    t = pl.program_id(1)
    nt = pl.num_programs(1)

    x = batch_ref[:, c, :]

    @pl.when(t == 0)
    def _init():
        minval[:, :] = jnp.full(minval.shape, jnp.inf, minval.dtype)
        minidx[:, :] = jnp.zeros(minidx.shape, minidx.dtype)

    k = keys_ref[0]
    xk = lax.dot_general(
        x, k, dimension_numbers=(((1,), (1,)), ((), ())),
        preferred_element_type=jnp.float32,
        precision=lax.Precision.DEFAULT)
    ones_row = jnp.ones((1, k.shape[1]), jnp.float32)
    kk = k * k
    kk_hi = lax.convert_element_type(
        lax.convert_element_type(kk, jnp.bfloat16), jnp.float32)
    kk_lo = kk - kk_hi
    dims = (((1,), (1,)), ((), ()))
    k2 = (lax.dot_general(ones_row, kk_hi, dimension_numbers=dims,
                          preferred_element_type=jnp.float32,
                          precision=lax.Precision.DEFAULT)
          + lax.dot_general(ones_row, kk_lo, dimension_numbers=dims,
                            preferred_element_type=jnp.float32,
                            precision=lax.Precision.DEFAULT))
    x2 = jnp.sum(x * x, axis=1, keepdims=True)
    d2 = jnp.maximum(x2 + (k2 - 2.0 * xk), 0.0)
    tmin = jnp.min(d2, axis=1, keepdims=True)
    targ = jnp.argmin(d2, axis=1).astype(jnp.int32)[:, None] + t * TK
    better = tmin < minval[:, :]
    minidx[:, :] = jnp.where(better, targ, minidx[:, :])
    minval[:, :] = jnp.where(better, tmin, minval[:, :])

    @pl.when(t == nt - 1)
    def _emit():
        B, C = out_ref.shape
        col = lax.broadcasted_iota(jnp.int32, (B, C), 1)
        out_ref[:, :] = jnp.where(col == c, minidx[:, :] + c * K,
                                  out_ref[:, :])


def _make_sc_gather(B, C, K, V):
    bpw = B // _NW
    mesh = plsc.VectorSubcoreMesh(core_axis_name="c", subcore_axis_name="s")

    @functools.partial(
        pl.kernel, mesh=mesh,
        out_type=jax.ShapeDtypeStruct((B, V), jnp.float32),
        scratch_types=[
            pltpu.VMEM((bpw * C,), jnp.int32),
            pltpu.VMEM((bpw * C, V), jnp.float32),
            pltpu.VMEM((bpw, V), jnp.float32),
            pltpu.SemaphoreType.DMA,
        ],
    )
    def sc_gather(idx_hbm, values_hbm, out_hbm, idx_list, rows, acc, sem):
        wid = lax.axis_index("s") * _NC + lax.axis_index("c")
        base = wid * bpw
        pltpu.sync_copy(idx_hbm.at[pl.ds(base * C, bpw * C)], idx_list)
        pltpu.async_copy(values_hbm.at[idx_list], rows, sem).wait()
        inv_c = 1.0 / C
        for j in range(bpw):
            for t in range(V // _L):
                s = rows[j * C + 0, pl.ds(t * _L, _L)]
                for r in range(1, C):
                    s = s + rows[j * C + r, pl.ds(t * _L, _L)]
                acc[j, pl.ds(t * _L, _L)] = s * inv_c
        pltpu.sync_copy(acc, out_hbm.at[pl.ds(base, bpw)])

    return sc_gather


def kernel(batch, keys, values):
    B, C, D = batch.shape
    _, K, V = values.shape

    TK = 4096
    flat_idx = pl.pallas_call(
        functools.partial(_dist_argmin_body, K, TK),
        grid=(C, K // TK),
        in_specs=[
            pl.BlockSpec((B, C, D), lambda c, t: (0, 0, 0)),
            pl.BlockSpec((1, TK, D), lambda c, t: (c, t, 0)),
        ],
        out_specs=pl.BlockSpec((B, C), lambda c, t: (0, 0)),
        out_shape=jax.ShapeDtypeStruct((B, C), jnp.int32),
        scratch_shapes=[
            pltpu.VMEM((B, 1), jnp.float32),
            pltpu.VMEM((B, 1), jnp.int32),
        ],
        compiler_params=pltpu.CompilerParams(
            dimension_semantics=("arbitrary", "arbitrary")),
    )(batch, keys)

    sc_gather = _make_sc_gather(B, C, K, V)
    return sc_gather(flat_idx.reshape(B * C), values.reshape(C * K, V))


# speedup vs baseline: 1.0516x; 1.0516x over previous
import functools

import jax
import jax.numpy as jnp
from jax import lax
from jax.experimental import pallas as pl
from jax.experimental.pallas import tpu as pltpu
from jax.experimental.pallas import tpu_sc as plsc

_NC, _NS, _L = 2, 16, 16
_NW = _NC * _NS


def _dist_argmin_body(K, TK, batch_ref, keys_ref, out_ref, minval, minidx):
    c = pl.program_id(0)
    t = pl.program_id(1)
    nt = pl.num_programs(1)

    x = batch_ref[:, c, :]

    @pl.when(t == 0)
    def _init():
        minval[:, :] = jnp.full(minval.shape, jnp.inf, minval.dtype)
        minidx[:, :] = jnp.zeros(minidx.shape, minidx.dtype)

    k = keys_ref[0]
    xk = lax.dot_general(
        x, k, dimension_numbers=(((1,), (1,)), ((), ())),
        preferred_element_type=jnp.float32,
        precision=lax.Precision.DEFAULT)
    ones_row = jnp.ones((1, k.shape[1]), jnp.float32)
    kk = k * k
    kk_hi = lax.convert_element_type(
        lax.convert_element_type(kk, jnp.bfloat16), jnp.float32)
    kk_lo = kk - kk_hi
    dims = (((1,), (1,)), ((), ()))
    k2 = (lax.dot_general(ones_row, kk_hi, dimension_numbers=dims,
                          preferred_element_type=jnp.float32,
                          precision=lax.Precision.DEFAULT)
          + lax.dot_general(ones_row, kk_lo, dimension_numbers=dims,
                            preferred_element_type=jnp.float32,
                            precision=lax.Precision.DEFAULT))
    x2 = jnp.sum(x * x, axis=1, keepdims=True)
    d2 = jnp.maximum(x2 + (k2 - 2.0 * xk), 0.0)
    tmin = jnp.min(d2, axis=1, keepdims=True)
    targ = jnp.argmin(d2, axis=1).astype(jnp.int32)[:, None] + t * TK
    better = tmin < minval[:, :]
    minidx[:, :] = jnp.where(better, targ, minidx[:, :])
    minval[:, :] = jnp.where(better, tmin, minval[:, :])

    @pl.when(t == nt - 1)
    def _emit():
        B, C = out_ref.shape
        col = lax.broadcasted_iota(jnp.int32, (B, C), 1)
        out_ref[:, :] = jnp.where(col == c, minidx[:, :] + c * K,
                                  out_ref[:, :])


def _make_sc_gather(B, C, K, V):
    bpw = B // _NW
    mesh = plsc.VectorSubcoreMesh(core_axis_name="c", subcore_axis_name="s")

    @functools.partial(
        pl.kernel, mesh=mesh,
        out_type=jax.ShapeDtypeStruct((B, V), jnp.float32),
        scratch_types=[
            pltpu.VMEM((bpw * C,), jnp.int32),
            pltpu.VMEM((bpw * C, V), jnp.float32),
            pltpu.VMEM((bpw, V), jnp.float32),
            pltpu.SemaphoreType.DMA,
        ],
    )
    def sc_gather(idx_hbm, values_hbm, out_hbm, idx_list, rows, acc, sem):
        wid = lax.axis_index("s") * _NC + lax.axis_index("c")
        base = wid * bpw
        pltpu.sync_copy(idx_hbm.at[pl.ds(base * C, bpw * C)], idx_list)
        pltpu.async_copy(values_hbm.at[idx_list], rows, sem).wait()
        inv_c = 1.0 / C
        for j in range(bpw):
            for t in range(V // _L):
                s = rows[j * C + 0, pl.ds(t * _L, _L)]
                for r in range(1, C):
                    s = s + rows[j * C + r, pl.ds(t * _L, _L)]
                acc[j, pl.ds(t * _L, _L)] = s * inv_c
        pltpu.sync_copy(acc, out_hbm.at[pl.ds(base, bpw)])

    return sc_gather


def kernel(batch, keys, values):
    B, C, D = batch.shape
    _, K, V = values.shape

    TK = 8192
    flat_idx = pl.pallas_call(
        functools.partial(_dist_argmin_body, K, TK),
        grid=(C, K // TK),
        in_specs=[
            pl.BlockSpec((B, C, D), lambda c, t: (0, 0, 0)),
            pl.BlockSpec((1, TK, D), lambda c, t: (c, t, 0)),
        ],
        out_specs=pl.BlockSpec((B, C), lambda c, t: (0, 0)),
        out_shape=jax.ShapeDtypeStruct((B, C), jnp.int32),
        scratch_shapes=[
            pltpu.VMEM((B, 1), jnp.float32),
            pltpu.VMEM((B, 1), jnp.int32),
        ],
        compiler_params=pltpu.CompilerParams(
            dimension_semantics=("arbitrary", "arbitrary")),
    )(batch, keys)

    sc_gather = _make_sc_gather(B, C, K, V)
    return sc_gather(flat_idx.reshape(B * C), values.reshape(C * K, V))


# speedup vs baseline: 1.0649x; 1.0126x over previous
import functools

import jax
import jax.numpy as jnp
from jax import lax
from jax.experimental import pallas as pl
from jax.experimental.pallas import tpu as pltpu
from jax.experimental.pallas import tpu_sc as plsc

_NC, _NS, _L = 2, 16, 16
_NW = _NC * _NS


def _dist_argmin_body(K, TK, batch_ref, keys_ref, out_ref, minval, minidx):
    c = pl.program_id(0)
    t = pl.program_id(1)
    nt = pl.num_programs(1)

    x = batch_ref[:, c, :]

    @pl.when(t == 0)
    def _init():
        minval[:, :] = jnp.full(minval.shape, jnp.inf, minval.dtype)
        minidx[:, :] = jnp.zeros(minidx.shape, minidx.dtype)

    k = keys_ref[0]
    xk = lax.dot_general(
        x, k, dimension_numbers=(((1,), (1,)), ((), ())),
        preferred_element_type=jnp.float32,
        precision=lax.Precision.DEFAULT)
    ones_row = jnp.ones((1, k.shape[1]), jnp.float32)
    kk = k * k
    kk_hi = lax.convert_element_type(
        lax.convert_element_type(kk, jnp.bfloat16), jnp.float32)
    kk_lo = kk - kk_hi
    dims = (((1,), (1,)), ((), ()))
    k2 = (lax.dot_general(ones_row, kk_hi, dimension_numbers=dims,
                          preferred_element_type=jnp.float32,
                          precision=lax.Precision.DEFAULT)
          + lax.dot_general(ones_row, kk_lo, dimension_numbers=dims,
                            preferred_element_type=jnp.float32,
                            precision=lax.Precision.DEFAULT))
    x2 = jnp.sum(x * x, axis=1, keepdims=True)
    d2 = jnp.maximum(x2 + (k2 - 2.0 * xk), 0.0)
    tmin = jnp.min(d2, axis=1, keepdims=True)
    targ = jnp.argmin(d2, axis=1).astype(jnp.int32)[:, None] + t * TK
    better = tmin < minval[:, :]
    minidx[:, :] = jnp.where(better, targ, minidx[:, :])
    minval[:, :] = jnp.where(better, tmin, minval[:, :])

    @pl.when(t == nt - 1)
    def _emit():
        B, C = out_ref.shape
        col = lax.broadcasted_iota(jnp.int32, (B, C), 1)
        out_ref[:, :] = jnp.where(col == c, minidx[:, :] + c * K,
                                  out_ref[:, :])


def _make_sc_gather(B, C, K, V):
    bpw = B // _NW
    mesh = plsc.VectorSubcoreMesh(core_axis_name="c", subcore_axis_name="s")

    @functools.partial(
        pl.kernel, mesh=mesh,
        out_type=jax.ShapeDtypeStruct((B, V), jnp.float32),
        scratch_types=[
            pltpu.VMEM((bpw, C), jnp.int32),
            pltpu.VMEM((bpw * C,), jnp.int32),
            pltpu.VMEM((bpw * C, V), jnp.float32),
            pltpu.VMEM((bpw, V), jnp.float32),
            pltpu.SemaphoreType.DMA,
        ],
    )
    def sc_gather(idx_hbm, values_hbm, out_hbm, idx2, idx_list, rows, acc,
                  sem):
        wid = lax.axis_index("s") * _NC + lax.axis_index("c")
        base = wid * bpw
        pltpu.sync_copy(idx_hbm.at[pl.ds(base, bpw)], idx2)
        for j in range(bpw):
            idx_list[pl.ds(j * C, C)] = idx2[j, :]
        pltpu.async_copy(values_hbm.at[idx_list], rows, sem).wait()
        inv_c = 1.0 / C
        for j in range(bpw):
            for t in range(V // _L):
                s = rows[j * C + 0, pl.ds(t * _L, _L)]
                for r in range(1, C):
                    s = s + rows[j * C + r, pl.ds(t * _L, _L)]
                acc[j, pl.ds(t * _L, _L)] = s * inv_c
        pltpu.sync_copy(acc, out_hbm.at[pl.ds(base, bpw)])

    return sc_gather


def kernel(batch, keys, values):
    B, C, D = batch.shape
    _, K, V = values.shape

    TK = 8192
    flat_idx = pl.pallas_call(
        functools.partial(_dist_argmin_body, K, TK),
        grid=(C, K // TK),
        in_specs=[
            pl.BlockSpec((B, C, D), lambda c, t: (0, 0, 0)),
            pl.BlockSpec((1, TK, D), lambda c, t: (c, t, 0)),
        ],
        out_specs=pl.BlockSpec((B, C), lambda c, t: (0, 0)),
        out_shape=jax.ShapeDtypeStruct((B, C), jnp.int32),
        scratch_shapes=[
            pltpu.VMEM((B, 1), jnp.float32),
            pltpu.VMEM((B, 1), jnp.int32),
        ],
        compiler_params=pltpu.CompilerParams(
            dimension_semantics=("arbitrary", "arbitrary")),
    )(batch, keys)

    sc_gather = _make_sc_gather(B, C, K, V)
    return sc_gather(flat_idx, values.reshape(C * K, V))
